# baseline (speedup 1.0000x reference)
import functools

import jax
import jax.numpy as jnp
from jax import lax
from jax.experimental import pallas as pl
from jax.experimental.pallas import tpu as pltpu
from jax.experimental.pallas import tpu_sc as plsc

B = 4096
HIST = 50
D = 64
DP = 128
NCOLS = 6
ELEM = D * NCOLS
VBIG = 100000
VSMALL = 1000

_INFO = plsc.get_sparse_core_info()
_NC = _INFO.num_cores
_NS = _INFO.num_subcores
_NW = _NC * _NS
_BPW = B // _NW
_CHUNK = 128
_NCHUNK = (_BPW * HIST) // _CHUNK

_sc_mesh = plsc.VectorSubcoreMesh(core_axis_name="c", subcore_axis_name="s")


def _worker(base_unit):
    sid = lax.axis_index("s")
    wid = sid * _NC + lax.axis_index("c")
    return sid, wid, wid * base_unit


@functools.partial(
    pl.kernel,
    out_type=jax.ShapeDtypeStruct((B, DP), jnp.float32),
    mesh=_sc_mesh,
    scratch_types=[
        pltpu.VMEM((_NCHUNK, _CHUNK), jnp.int32),
        pltpu.VMEM((_NCHUNK, _CHUNK), jnp.int32),
        pltpu.VMEM((2, _CHUNK, DP), jnp.float32),
        pltpu.VMEM_SHARED((_NS * _BPW, DP), jnp.float32),
        pltpu.SemaphoreType.DMA,
        pltpu.SemaphoreType.DMA,
        pltpu.SemaphoreType.DMA,
        pltpu.SemaphoreType.DMA,
        pltpu.SemaphoreType.DMA,
    ],
    compiler_params=pltpu.CompilerParams(use_tc_tiling_on_sc=True),
)
def _sc_hist(hist_idx, seg_hbm, zeros_hbm, tabp, out_hist,
             hidx_v, seg_v, hbuf_v, acc_sh,
             sem_m, sem_g0, sem_g1, sem_s0, sem_s1):
    sid, wid, base = _worker(_BPW)
    acc_slot = acc_sh.at[pl.ds(sid * _BPW, _BPW)]
    sem_g = (sem_g0, sem_g1)
    sem_s = (sem_s0, sem_s1)

    stage = [pltpu.async_copy(hist_idx.at[wid], hidx_v, sem_m),
             pltpu.async_copy(seg_hbm.at[sid], seg_v, sem_m),
             pltpu.async_copy(zeros_hbm, acc_slot, sem_m)]
    for c in stage:
        c.wait()

    def h_gather(c, b):
        pltpu.async_copy(tabp.at[hidx_v.at[c]], hbuf_v.at[b], sem_g[b])

    def h_scatter(c, b):
        pltpu.async_copy(hbuf_v.at[b], acc_sh.at[seg_v.at[c]], sem_s[b],
                         add=True)

    def wait_gather(b):
        pltpu.make_async_copy(tabp.at[hidx_v.at[0]], hbuf_v.at[b],
                              sem_g[b]).wait()

    def wait_scatter(b):
        pltpu.make_async_copy(hbuf_v.at[b], acc_sh.at[seg_v.at[0]],
                              sem_s[b]).wait()

    h_gather(0, 0)
    h_gather(1, 1)

    def group(j, carry):
        for b in range(2):
            wait_gather(b)
            h_scatter(2 * j + b, b)

        @pl.when(j < _NCHUNK // 2 - 1)
        def _():
            for b in range(2):
                wait_scatter(b)
                h_gather(2 * j + 2 + b, b)

        return carry

    lax.fori_loop(0, _NCHUNK // 2, group, 0)
    for b in range(2):
        wait_scatter(b)
    pltpu.sync_copy(acc_slot, out_hist.at[pl.ds(base, _BPW)])


@functools.partial(
    pl.kernel,
    out_type=[jax.ShapeDtypeStruct((B, DP), jnp.float32) for _ in range(5)],
    mesh=_sc_mesh,
    scratch_types=[
        pltpu.VMEM((5, _BPW), jnp.int32),
        pltpu.VMEM((5, _BPW, DP), jnp.float32),
        pltpu.SemaphoreType.DMA,
        pltpu.SemaphoreType.DMA,
        pltpu.SemaphoreType.DMA,
    ],
    compiler_params=pltpu.CompilerParams(use_tc_tiling_on_sc=True),
)
def _sc_singles(idx_user, idx_item, idx_cate, idx_hour, idx_device,
                tab_user, tab_item, tab_cate, tab_hour, tab_device,
                out_user, out_item, out_cate, out_hour, out_device,
                fidx_v, frows_v, sem_m, sem_f, sem_o):
    _, _, base = _worker(_BPW)
    idxs = (idx_user, idx_item, idx_cate, idx_hour, idx_device)
    tabs = (tab_user, tab_item, tab_cate, tab_hour, tab_device)
    outs = (out_user, out_item, out_cate, out_hour, out_device)

    stage = [pltpu.async_copy(idx_hbm.at[pl.ds(base, _BPW)], fidx_v.at[k],
                              sem_m)
             for k, idx_hbm in enumerate(idxs)]
    for c in stage:
        c.wait()
    fg = [pltpu.async_copy(tabs[k].at[fidx_v.at[k]], frows_v.at[k], sem_f)
          for k in range(5)]
    for c in fg:
        c.wait()
    outw = [pltpu.async_copy(frows_v.at[k], out_hbm.at[pl.ds(base, _BPW)],
                             sem_o)
            for k, out_hbm in enumerate(outs)]
    for c in outw:
        c.wait()


def _mlp_body(eu, ei, ec, eh, ed, ehist, w1, b1, w2, b2, w3, b3, out):
    x = jnp.concatenate(
        [eu[:, :D], ei[:, :D], ec[:, :D], eh[:, :D], ed[:, :D],
         ehist[:, :D] * (1.0 / HIST)], axis=1)
    h = jax.nn.relu(jnp.dot(x, w1[...], preferred_element_type=jnp.float32)
                    + b1[...])
    h = jax.nn.relu(jnp.dot(h, w2[...], preferred_element_type=jnp.float32)
                    + b2[...])
    out[...] = jax.nn.sigmoid(
        jnp.dot(h, w3[...], preferred_element_type=jnp.float32) + b3[...])


def _mlp(feats, W1, b1, W2, b2, W3, b3):
    BB = 512
    grid = (B // BB,)
    feat_spec = pl.BlockSpec((BB, DP), lambda i: (i, 0))
    full = lambda shape: pl.BlockSpec(shape, lambda i: tuple(0 for _ in shape))
    return pl.pallas_call(
        _mlp_body,
        grid=grid,
        in_specs=[feat_spec] * 6 + [
            full((ELEM, ELEM)), full((1, ELEM)),
            full((ELEM, ELEM)), full((1, ELEM)),
            full((ELEM, 1)), full((1, 1)),
        ],
        out_specs=pl.BlockSpec((BB, 1), lambda i: (i, 0)),
        out_shape=jax.ShapeDtypeStruct((B, 1), jnp.float32),
    )(*feats, W1, b1.reshape(1, ELEM), W2, b2.reshape(1, ELEM),
      W3, b3.reshape(1, 1))


def kernel(idx_user, idx_item, idx_cate, idx_hour, idx_device, idx_hist,
           tab_user, tab_item, tab_cate, tab_hour, tab_device, tab_hist,
           W1, b1, W2, b2, W3, b3):
    i32 = lambda a: a.astype(jnp.int32)

    pad = lambda t: jnp.pad(t, ((0, 0), (0, DP - D)))
    histp = pad(tab_hist)
    tab_user = lax.optimization_barrier((tab_user, histp))[0]
    tab_item = lax.optimization_barrier((tab_item, histp))[0]
    tab_cate = lax.optimization_barrier((tab_cate, histp))[0]
    userp, itemp, catep = pad(tab_user), pad(tab_item), pad(tab_cate)
    hourp, devp = pad(tab_hour), pad(tab_device)

    hist_idx = i32(idx_hist).reshape(_NW, _NCHUNK, _CHUNK)
    seg_local = (jnp.arange(_BPW * HIST, dtype=jnp.int32) // HIST).reshape(
        1, _NCHUNK, _CHUNK)
    seg = seg_local + (jnp.arange(_NS, dtype=jnp.int32) * _BPW).reshape(
        _NS, 1, 1)
    zeros = jnp.zeros((_BPW, DP), jnp.float32)

    ehist = _sc_hist(hist_idx, seg, zeros, histp)
    singles = _sc_singles(
        i32(idx_user), i32(idx_item), i32(idx_cate),
        i32(idx_hour), i32(idx_device),
        userp, itemp, catep, hourp, devp)
    return _mlp(list(singles) + [ehist], W1, b1, W2, b2, W3, b3)


# speedup vs baseline: 1.1234x; 1.1234x over previous
import functools

import jax
import jax.numpy as jnp
from jax import lax
from jax.experimental import pallas as pl
from jax.experimental.pallas import tpu as pltpu
from jax.experimental.pallas import tpu_sc as plsc

B = 4096
HIST = 50
D = 64
DP = 128
NCOLS = 6
ELEM = D * NCOLS
VBIG = 100000
VSMALL = 1000

_INFO = plsc.get_sparse_core_info()
_NC = _INFO.num_cores
_NS = _INFO.num_subcores
_NW = _NC * _NS
_BPW = B // _NW
_CHUNK = 128
_NCHUNK = (_BPW * HIST) // _CHUNK
_NBUF = 5

_sc_mesh = plsc.VectorSubcoreMesh(core_axis_name="c", subcore_axis_name="s")


def _worker(base_unit):
    sid = lax.axis_index("s")
    wid = sid * _NC + lax.axis_index("c")
    return sid, wid, wid * base_unit


@functools.partial(
    pl.kernel,
    out_type=jax.ShapeDtypeStruct((B, DP), jnp.float32),
    mesh=_sc_mesh,
    scratch_types=[
        pltpu.VMEM((_NCHUNK, _CHUNK), jnp.int32),
        pltpu.VMEM((_NCHUNK, _CHUNK), jnp.int32),
        pltpu.VMEM((_NBUF, _CHUNK, DP), jnp.float32),
        pltpu.VMEM_SHARED((_NS * _BPW, DP), jnp.float32),
        pltpu.SemaphoreType.DMA,
    ] + [pltpu.SemaphoreType.DMA] * (2 * _NBUF),
    compiler_params=pltpu.CompilerParams(use_tc_tiling_on_sc=True),
)
def _sc_hist(hist_idx, seg_hbm, zeros_hbm, tabp, out_hist,
             hidx_v, seg_v, hbuf_v, acc_sh, sem_m, *sems):
    sid, wid, base = _worker(_BPW)
    acc_slot = acc_sh.at[pl.ds(sid * _BPW, _BPW)]
    sem_g = sems[:_NBUF]
    sem_s = sems[_NBUF:]

    stage = [pltpu.async_copy(hist_idx.at[wid], hidx_v, sem_m),
             pltpu.async_copy(seg_hbm.at[sid], seg_v, sem_m),
             pltpu.async_copy(zeros_hbm, acc_slot, sem_m)]
    for c in stage:
        c.wait()

    def h_gather(c, b):
        pltpu.async_copy(tabp.at[hidx_v.at[c]], hbuf_v.at[b], sem_g[b])

    def h_scatter(c, b):
        pltpu.async_copy(hbuf_v.at[b], acc_sh.at[seg_v.at[c]], sem_s[b],
                         add=True)

    def wait_gather(b):
        pltpu.make_async_copy(tabp.at[hidx_v.at[0]], hbuf_v.at[b],
                              sem_g[b]).wait()

    def wait_scatter(b):
        pltpu.make_async_copy(hbuf_v.at[b], acc_sh.at[seg_v.at[0]],
                              sem_s[b]).wait()

    for b in range(_NBUF):
        h_gather(b, b)

    def group(j, carry):
        for b in range(_NBUF):
            wait_gather(b)
            h_scatter(_NBUF * j + b, b)

        @pl.when(j < _NCHUNK // _NBUF - 1)
        def _():
            for b in range(_NBUF):
                wait_scatter(b)
                h_gather(_NBUF * (j + 1) + b, b)

        return carry

    lax.fori_loop(0, _NCHUNK // _NBUF, group, 0)
    for b in range(_NBUF):
        wait_scatter(b)
    pltpu.sync_copy(acc_slot, out_hist.at[pl.ds(base, _BPW)])


@functools.partial(
    pl.kernel,
    out_type=[jax.ShapeDtypeStruct((B, DP), jnp.float32) for _ in range(5)],
    mesh=_sc_mesh,
    scratch_types=[
        pltpu.VMEM((5, _BPW), jnp.int32),
        pltpu.VMEM((5, _BPW, DP), jnp.float32),
        pltpu.SemaphoreType.DMA,
        pltpu.SemaphoreType.DMA,
        pltpu.SemaphoreType.DMA,
    ],
    compiler_params=pltpu.CompilerParams(use_tc_tiling_on_sc=True),
)
def _sc_singles(idx_user, idx_item, idx_cate, idx_hour, idx_device,
                tab_user, tab_item, tab_cate, tab_hour, tab_device,
                out_user, out_item, out_cate, out_hour, out_device,
                fidx_v, frows_v, sem_m, sem_f, sem_o):
    _, _, base = _worker(_BPW)
    idxs = (idx_user, idx_item, idx_cate, idx_hour, idx_device)
    tabs = (tab_user, tab_item, tab_cate, tab_hour, tab_device)
    outs = (out_user, out_item, out_cate, out_hour, out_device)

    stage = [pltpu.async_copy(idx_hbm.at[pl.ds(base, _BPW)], fidx_v.at[k],
                              sem_m)
             for k, idx_hbm in enumerate(idxs)]
    for c in stage:
        c.wait()
    fg = [pltpu.async_copy(tabs[k].at[fidx_v.at[k]], frows_v.at[k], sem_f)
          for k in range(5)]
    for c in fg:
        c.wait()
    outw = [pltpu.async_copy(frows_v.at[k], out_hbm.at[pl.ds(base, _BPW)],
                             sem_o)
            for k, out_hbm in enumerate(outs)]
    for c in outw:
        c.wait()


def _mlp_body(eu, ei, ec, eh, ed, ehist, w1, b1, w2, b2, w3, b3, out):
    x = jnp.concatenate(
        [eu[:, :D], ei[:, :D], ec[:, :D], eh[:, :D], ed[:, :D],
         ehist[:, :D] * (1.0 / HIST)], axis=1)
    h = jax.nn.relu(jnp.dot(x, w1[...], preferred_element_type=jnp.float32)
                    + b1[...])
    h = jax.nn.relu(jnp.dot(h, w2[...], preferred_element_type=jnp.float32)
                    + b2[...])
    out[...] = jax.nn.sigmoid(
        jnp.dot(h, w3[...], preferred_element_type=jnp.float32) + b3[...])


def _mlp(feats, W1, b1, W2, b2, W3, b3):
    BB = 512
    grid = (B // BB,)
    feat_spec = pl.BlockSpec((BB, DP), lambda i: (i, 0))
    full = lambda shape: pl.BlockSpec(shape, lambda i: tuple(0 for _ in shape))
    return pl.pallas_call(
        _mlp_body,
        grid=grid,
        in_specs=[feat_spec] * 6 + [
            full((ELEM, ELEM)), full((1, ELEM)),
            full((ELEM, ELEM)), full((1, ELEM)),
            full((ELEM, 1)), full((1, 1)),
        ],
        out_specs=pl.BlockSpec((BB, 1), lambda i: (i, 0)),
        out_shape=jax.ShapeDtypeStruct((B, 1), jnp.float32),
    )(*feats, W1, b1.reshape(1, ELEM), W2, b2.reshape(1, ELEM),
      W3, b3.reshape(1, 1))


def kernel(idx_user, idx_item, idx_cate, idx_hour, idx_device, idx_hist,
           tab_user, tab_item, tab_cate, tab_hour, tab_device, tab_hist,
           W1, b1, W2, b2, W3, b3):
    i32 = lambda a: a.astype(jnp.int32)

    pad = lambda t: jnp.pad(t, ((0, 0), (0, DP - D)))
    histp = pad(tab_hist)
    tab_user = lax.optimization_barrier((tab_user, histp))[0]
    userp, itemp, catep = pad(tab_user), pad(tab_item), pad(tab_cate)
    hourp, devp = pad(tab_hour), pad(tab_device)

    hist_idx = i32(idx_hist).reshape(_NW, _NCHUNK, _CHUNK)
    seg_local = (jnp.arange(_BPW * HIST, dtype=jnp.int32) // HIST).reshape(
        1, _NCHUNK, _CHUNK)
    seg = seg_local + (jnp.arange(_NS, dtype=jnp.int32) * _BPW).reshape(
        _NS, 1, 1)
    zeros = jnp.zeros((_BPW, DP), jnp.float32)

    ehist = _sc_hist(hist_idx, seg, zeros, histp)
    singles = _sc_singles(
        i32(idx_user), i32(idx_item), i32(idx_cate),
        i32(idx_hour), i32(idx_device),
        userp, itemp, catep, hourp, devp)
    return _mlp(list(singles) + [ehist], W1, b1, W2, b2, W3, b3)


# speedup vs baseline: 1.1290x; 1.0050x over previous
import functools

import jax
import jax.numpy as jnp
from jax import lax
from jax.experimental import pallas as pl
from jax.experimental.pallas import tpu as pltpu
from jax.experimental.pallas import tpu_sc as plsc

B = 4096
HIST = 50
D = 64
DP = 128
NCOLS = 6
ELEM = D * NCOLS
VBIG = 100000
VSMALL = 1000

_INFO = plsc.get_sparse_core_info()
_NC = _INFO.num_cores
_NS = _INFO.num_subcores
_NW = _NC * _NS
_BPW = B // _NW
_CHUNK = 128
_NCHUNK = (_BPW * HIST) // _CHUNK
_NBUF = 5

_sc_mesh = plsc.VectorSubcoreMesh(core_axis_name="c", subcore_axis_name="s")


def _worker(base_unit):
    sid = lax.axis_index("s")
    wid = sid * _NC + lax.axis_index("c")
    return sid, wid, wid * base_unit


@functools.partial(
    pl.kernel,
    out_type=jax.ShapeDtypeStruct((B, DP), jnp.float32),
    mesh=_sc_mesh,
    scratch_types=[
        pltpu.VMEM((_NCHUNK, _CHUNK), jnp.int32),
        pltpu.VMEM((_NCHUNK, _CHUNK), jnp.int32),
        pltpu.VMEM((_NBUF, _CHUNK, DP), jnp.float32),
        pltpu.VMEM_SHARED((_NS * _BPW, DP), jnp.float32),
        pltpu.SemaphoreType.DMA,
    ] + [pltpu.SemaphoreType.DMA] * (2 * _NBUF),
    compiler_params=pltpu.CompilerParams(use_tc_tiling_on_sc=True),
)
def _sc_hist(hist_idx, seg_hbm, zeros_hbm, tabp, out_hist,
             hidx_v, seg_v, hbuf_v, acc_sh, sem_m, *sems):
    sid, wid, base = _worker(_BPW)
    acc_slot = acc_sh.at[pl.ds(sid * _BPW, _BPW)]
    sem_g = sems[:_NBUF]
    sem_s = sems[_NBUF:]

    stage = [pltpu.async_copy(hist_idx.at[wid], hidx_v, sem_m),
             pltpu.async_copy(seg_hbm.at[sid], seg_v, sem_m),
             pltpu.async_copy(zeros_hbm, acc_slot, sem_m)]
    for c in stage:
        c.wait()

    def h_gather(c, b):
        pltpu.async_copy(tabp.at[hidx_v.at[c]], hbuf_v.at[b], sem_g[b])

    def h_scatter(c, b):
        pltpu.async_copy(hbuf_v.at[b], acc_sh.at[seg_v.at[c]], sem_s[b],
                         add=True)

    def wait_gather(b):
        pltpu.make_async_copy(tabp.at[hidx_v.at[0]], hbuf_v.at[b],
                              sem_g[b]).wait()

    def wait_scatter(b):
        pltpu.make_async_copy(hbuf_v.at[b], acc_sh.at[seg_v.at[0]],
                              sem_s[b]).wait()

    for b in range(_NBUF):
        h_gather(b, b)

    def group(j, carry):
        for b in range(_NBUF):
            wait_gather(b)
            h_scatter(_NBUF * j + b, b)

        @pl.when(j < _NCHUNK // _NBUF - 1)
        def _():
            for b in range(_NBUF):
                wait_scatter(b)
                h_gather(_NBUF * (j + 1) + b, b)

        return carry

    lax.fori_loop(0, _NCHUNK // _NBUF, group, 0)
    for b in range(_NBUF):
        wait_scatter(b)
    pltpu.sync_copy(acc_slot, out_hist.at[pl.ds(base, _BPW)])


@functools.partial(
    pl.kernel,
    out_type=[jax.ShapeDtypeStruct((B, DP), jnp.float32) for _ in range(5)],
    mesh=_sc_mesh,
    scratch_types=[
        pltpu.VMEM((5, _BPW), jnp.int32),
        pltpu.VMEM((5, _BPW, DP), jnp.float32),
        pltpu.SemaphoreType.DMA,
        pltpu.SemaphoreType.DMA,
        pltpu.SemaphoreType.DMA,
    ],
    compiler_params=pltpu.CompilerParams(use_tc_tiling_on_sc=True),
)
def _sc_singles(idx_user, idx_item, idx_cate, idx_hour, idx_device,
                tab_user, tab_item, tab_cate, tab_hour, tab_device,
                out_user, out_item, out_cate, out_hour, out_device,
                fidx_v, frows_v, sem_m, sem_f, sem_o):
    _, _, base = _worker(_BPW)
    idxs = (idx_user, idx_item, idx_cate, idx_hour, idx_device)
    tabs = (tab_user, tab_item, tab_cate, tab_hour, tab_device)
    outs = (out_user, out_item, out_cate, out_hour, out_device)

    stage = [pltpu.async_copy(idx_hbm.at[pl.ds(base, _BPW)], fidx_v.at[k],
                              sem_m)
             for k, idx_hbm in enumerate(idxs)]
    for c in stage:
        c.wait()
    fg = [pltpu.async_copy(tabs[k].at[fidx_v.at[k]], frows_v.at[k], sem_f)
          for k in range(5)]
    for c in fg:
        c.wait()
    outw = [pltpu.async_copy(frows_v.at[k], out_hbm.at[pl.ds(base, _BPW)],
                             sem_o)
            for k, out_hbm in enumerate(outs)]
    for c in outw:
        c.wait()


def _mlp_body(eu, ei, ec, eh, ed, ehist, w1, b1, w2, b2, w3, b3, out):
    x = jnp.concatenate(
        [eu[:, :D], ei[:, :D], ec[:, :D], eh[:, :D], ed[:, :D],
         ehist[:, :D] * (1.0 / HIST)], axis=1)
    h = jax.nn.relu(jnp.dot(x, w1[...], preferred_element_type=jnp.float32)
                    + b1[...])
    h = jax.nn.relu(jnp.dot(h, w2[...], preferred_element_type=jnp.float32)
                    + b2[...])
    out[...] = jax.nn.sigmoid(
        jnp.dot(h, w3[...], preferred_element_type=jnp.float32) + b3[...])


def _mlp(feats, W1, b1, W2, b2, W3, b3):
    BB = 512
    grid = (B // BB,)
    feat_spec = pl.BlockSpec((BB, DP), lambda i: (i, 0))
    full = lambda shape: pl.BlockSpec(shape, lambda i: tuple(0 for _ in shape))
    return pl.pallas_call(
        _mlp_body,
        grid=grid,
        in_specs=[feat_spec] * 6 + [
            full((ELEM, ELEM)), full((1, ELEM)),
            full((ELEM, ELEM)), full((1, ELEM)),
            full((ELEM, 1)), full((1, 1)),
        ],
        out_specs=pl.BlockSpec((BB, 1), lambda i: (i, 0)),
        out_shape=jax.ShapeDtypeStruct((B, 1), jnp.float32),
    )(*feats, W1, b1.reshape(1, ELEM), W2, b2.reshape(1, ELEM),
      W3, b3.reshape(1, 1))


def kernel(idx_user, idx_item, idx_cate, idx_hour, idx_device, idx_hist,
           tab_user, tab_item, tab_cate, tab_hour, tab_device, tab_hist,
           W1, b1, W2, b2, W3, b3):
    i32 = lambda a: a.astype(jnp.int32)

    def pad(t):
        z = jnp.zeros((t.shape[0], DP - D), jnp.float32)
        return jnp.concatenate([t, z], axis=1)

    histp = pad(tab_hist)
    tab_user = lax.optimization_barrier((tab_user, histp))[0]
    userp, itemp, catep = pad(tab_user), pad(tab_item), pad(tab_cate)
    hourp, devp = pad(tab_hour), pad(tab_device)

    hist_idx = i32(idx_hist).reshape(_NW, _NCHUNK, _CHUNK)
    seg_local = (jnp.arange(_BPW * HIST, dtype=jnp.int32) // HIST).reshape(
        1, _NCHUNK, _CHUNK)
    seg = seg_local + (jnp.arange(_NS, dtype=jnp.int32) * _BPW).reshape(
        _NS, 1, 1)
    zeros = jnp.zeros((_BPW, DP), jnp.float32)

    ehist = _sc_hist(hist_idx, seg, zeros, histp)
    singles = _sc_singles(
        i32(idx_user), i32(idx_item), i32(idx_cate),
        i32(idx_hour), i32(idx_device),
        userp, itemp, catep, hourp, devp)
    return _mlp(list(singles) + [ehist], W1, b1, W2, b2, W3, b3)
